# baseline (speedup 1.0000x reference)
import functools

import jax
import jax.numpy as jnp
from jax.experimental import pallas as pl
from jax.experimental.pallas import tpu as pltpu

_VMEM_LIMIT = 60 * 1024 * 1024


def _pool_body(x1_ref, x2_ref, o1_ref, o2_ref, *, inv_s):
    o1_ref[0, 0] = jnp.sum(x1_ref[0], axis=0, keepdims=True) * inv_s
    o2_ref[0, 0] = jnp.sum(x2_ref[0], axis=0, keepdims=True) * inv_s


def _epilogue_body(s1_ref, s2_ref, ws1_ref, ws2_ref, wt1_ref, wt2_ref,
                   awb_ref, p0_ref, p1_ref):
    hp = jax.lax.Precision.DEFAULT
    dn = (((1,), (1,)), ((), ()))
    n_s = s1_ref.shape[1]
    a1 = s1_ref[:, 0, 0, :]
    a2 = s2_ref[:, 0, 0, :]
    for i in range(1, n_s):
        a1 = a1 + s1_ref[:, i, 0, :]
        a2 = a2 + s2_ref[:, i, 0, :]
    h1 = jax.lax.dot_general(a1, ws1_ref[...], dn, precision=hp,
                             preferred_element_type=jnp.float32)
    s_out = jax.lax.dot_general(h1, ws2_ref[...], dn, precision=hp,
                                preferred_element_type=jnp.float32)
    h2 = jax.lax.dot_general(a2, wt1_ref[...], dn, precision=hp,
                             preferred_element_type=jnp.float32)
    t_out = jax.lax.dot_general(h2, wt2_ref[...], dn, precision=hp,
                                preferred_element_type=jnp.float32)
    c0 = awb_ref[0, 0] - awb_ref[1, 0]
    c1 = awb_ref[0, 1] - awb_ref[1, 1]
    cb = awb_ref[0, 2] - awb_ref[1, 2]
    d = s_out * c0 + t_out * c1 + cb
    p0 = jax.nn.sigmoid(d)
    p0_ref[...] = p0
    p1_ref[...] = 1.0 - p0


def kernel(x1, x2, ws1, ws2, wt1, wt2, aw, ab):
    N, C, T, W, H = x1.shape
    S = T * W * H
    x1t = x1.transpose(0, 3, 4, 2, 1).reshape(N, S, C)
    x2t = x2.transpose(0, 3, 4, 2, 1).reshape(N, S, C)

    n_s = 2 if (S % 2 == 0 and (S // 2) % 8 == 0) else 1
    s_blk = S // n_s

    sums1, sums2 = pl.pallas_call(
        functools.partial(_pool_body, inv_s=1.0 / S),
        out_shape=[jax.ShapeDtypeStruct((N, n_s, 1, C), jnp.float32)] * 2,
        grid=(n_s, N),
        in_specs=[
            pl.BlockSpec((1, s_blk, C), lambda s, n: (n, s, 0)),
            pl.BlockSpec((1, s_blk, C), lambda s, n: (n, s, 0)),
        ],
        out_specs=[
            pl.BlockSpec((1, 1, 1, C), lambda s, n: (n, s, 0, 0)),
            pl.BlockSpec((1, 1, 1, C), lambda s, n: (n, s, 0, 0)),
        ],
        compiler_params=pltpu.CompilerParams(
            dimension_semantics=("parallel", "parallel"),
            vmem_limit_bytes=_VMEM_LIMIT,
        ),
        cost_estimate=pl.CostEstimate(
            flops=int(2 * N * C * S),
            transcendentals=0,
            bytes_accessed=int(2 * N * C * S * 4 + 2 * N * C * 4),
        ),
    )(x1t, x2t)

    awb = jnp.concatenate([aw, ab.reshape(2, 1)], axis=1)

    p0, p1 = pl.pallas_call(
        _epilogue_body,
        out_shape=[jax.ShapeDtypeStruct((N, C), jnp.float32)] * 2,
        in_specs=[
            pl.BlockSpec(sums1.shape, lambda: (0, 0, 0, 0)),
            pl.BlockSpec(sums2.shape, lambda: (0, 0, 0, 0)),
            pl.BlockSpec(ws1.shape, lambda: (0, 0)),
            pl.BlockSpec(ws2.shape, lambda: (0, 0)),
            pl.BlockSpec(wt1.shape, lambda: (0, 0)),
            pl.BlockSpec(wt2.shape, lambda: (0, 0)),
            pl.BlockSpec(memory_space=pltpu.SMEM),
        ],
        out_specs=[
            pl.BlockSpec((N, C), lambda: (0, 0)),
            pl.BlockSpec((N, C), lambda: (0, 0)),
        ],
        compiler_params=pltpu.CompilerParams(
            vmem_limit_bytes=_VMEM_LIMIT,
        ),
    )(sums1, sums2, ws1, ws2, wt1, wt2, awb)

    p = jnp.stack([p0, p1], axis=-1)
    return p.reshape(N, C, 2, 1, 1, 1)


# speedup vs baseline: 1.0054x; 1.0054x over previous
import functools

import jax
import jax.numpy as jnp
from jax.experimental import pallas as pl
from jax.experimental.pallas import tpu as pltpu

_VMEM_LIMIT = 60 * 1024 * 1024


def _pool_body(x1_ref, x2_ref, o1_ref, o2_ref, *, inv_s):
    o1_ref[0, 0] = jnp.sum(x1_ref[0], axis=0, keepdims=True) * inv_s
    o2_ref[0, 0] = jnp.sum(x2_ref[0], axis=0, keepdims=True) * inv_s


def _epilogue_body(s1_ref, s2_ref, ws1_ref, ws2_ref, wt1_ref, wt2_ref,
                   awb_ref, p0_ref, p1_ref):
    hp = jax.lax.Precision.DEFAULT
    dn = (((1,), (1,)), ((), ()))
    n_s = s1_ref.shape[1]
    a1 = s1_ref[:, 0, 0, :]
    a2 = s2_ref[:, 0, 0, :]
    for i in range(1, n_s):
        a1 = a1 + s1_ref[:, i, 0, :]
        a2 = a2 + s2_ref[:, i, 0, :]
    h1 = jax.lax.dot_general(a1, ws1_ref[...], dn, precision=hp,
                             preferred_element_type=jnp.float32)
    s_out = jax.lax.dot_general(h1, ws2_ref[...], dn, precision=hp,
                                preferred_element_type=jnp.float32)
    h2 = jax.lax.dot_general(a2, wt1_ref[...], dn, precision=hp,
                             preferred_element_type=jnp.float32)
    t_out = jax.lax.dot_general(h2, wt2_ref[...], dn, precision=hp,
                                preferred_element_type=jnp.float32)
    c0 = awb_ref[0, 0] - awb_ref[1, 0]
    c1 = awb_ref[0, 1] - awb_ref[1, 1]
    cb = awb_ref[0, 2] - awb_ref[1, 2]
    d = s_out * c0 + t_out * c1 + cb
    p0 = jax.nn.sigmoid(d)
    p0_ref[...] = p0
    p1_ref[...] = 1.0 - p0


def kernel(x1, x2, ws1, ws2, wt1, wt2, aw, ab):
    N, C, T, W, H = x1.shape
    S = T * W * H
    x1t = x1.transpose(0, 3, 4, 2, 1).reshape(N, S, C)
    x2t = x2.transpose(0, 3, 4, 2, 1).reshape(N, S, C)

    n_s = 2 if (S % 2 == 0 and (S // 2) % 8 == 0) else 1
    s_blk = S // n_s

    sums1, sums2 = pl.pallas_call(
        functools.partial(_pool_body, inv_s=1.0 / S),
        out_shape=[jax.ShapeDtypeStruct((N, n_s, 1, C), jnp.float32)] * 2,
        grid=(N, n_s),
        in_specs=[
            pl.BlockSpec((1, s_blk, C), lambda n, s: (n, s, 0)),
            pl.BlockSpec((1, s_blk, C), lambda n, s: (n, s, 0)),
        ],
        out_specs=[
            pl.BlockSpec((1, 1, 1, C), lambda n, s: (n, s, 0, 0)),
            pl.BlockSpec((1, 1, 1, C), lambda n, s: (n, s, 0, 0)),
        ],
        compiler_params=pltpu.CompilerParams(
            dimension_semantics=("parallel", "parallel"),
            vmem_limit_bytes=_VMEM_LIMIT,
        ),
        cost_estimate=pl.CostEstimate(
            flops=int(2 * N * C * S),
            transcendentals=0,
            bytes_accessed=int(2 * N * C * S * 4 + 2 * N * C * 4),
        ),
    )(x1t, x2t)

    awb = jnp.concatenate([aw, ab.reshape(2, 1)], axis=1)

    p0, p1 = pl.pallas_call(
        _epilogue_body,
        out_shape=[jax.ShapeDtypeStruct((N, C), jnp.float32)] * 2,
        in_specs=[
            pl.BlockSpec(sums1.shape, lambda: (0, 0, 0, 0)),
            pl.BlockSpec(sums2.shape, lambda: (0, 0, 0, 0)),
            pl.BlockSpec(ws1.shape, lambda: (0, 0)),
            pl.BlockSpec(ws2.shape, lambda: (0, 0)),
            pl.BlockSpec(wt1.shape, lambda: (0, 0)),
            pl.BlockSpec(wt2.shape, lambda: (0, 0)),
            pl.BlockSpec(memory_space=pltpu.SMEM),
        ],
        out_specs=[
            pl.BlockSpec((N, C), lambda: (0, 0)),
            pl.BlockSpec((N, C), lambda: (0, 0)),
        ],
        compiler_params=pltpu.CompilerParams(
            vmem_limit_bytes=_VMEM_LIMIT,
        ),
    )(sums1, sums2, ws1, ws2, wt1, wt2, awb)

    p = jnp.stack([p0, p1], axis=-1)
    return p.reshape(N, C, 2, 1, 1, 1)
